# Initial kernel scaffold; baseline (speedup 1.0000x reference)
import jax
import jax.numpy as jnp
from jax.experimental import pallas as pl


def kernel(x, adj, seg_index, Wl1, Wr1, a1, ln1_g, ln1_b, Wl2, Wr2, a2, ln2_g, ln2_b, cls_W, cls_b):
    raise NotImplementedError("write your pallas kernel here")


# speedup vs baseline: 8.5277x; 8.5277x over previous
import functools
import jax
import jax.numpy as jnp
from jax import lax
from jax.experimental import pallas as pl
from jax.experimental.pallas import tpu as pltpu
from jax.experimental.pallas import tpu_sc as plsc

N = 50000
E = 800000
NC = 2
NS = 16
R = N // NC
DRAIN = 1563
RT = NS * DRAIN
W_COL = 80


def _sc_gather(hl, hr, src, dst):
  CH = 40
  PERW = E // (NC * NS)
  ITERS = PERW // CH

  mesh = plsc.VectorSubcoreMesh(core_axis_name="c", subcore_axis_name="s")

  @functools.partial(
      pl.kernel,
      out_type=(jax.ShapeDtypeStruct((E, 64), jnp.float32),
                jax.ShapeDtypeStruct((E, 64), jnp.float32)),
      mesh=mesh,
      compiler_params=pltpu.CompilerParams(use_tc_tiling_on_sc=False),
      scratch_types=[
          pltpu.VMEM((CH,), jnp.int32),
          pltpu.VMEM((CH,), jnp.int32),
          pltpu.VMEM((CH, 64), jnp.float32),
          pltpu.VMEM((CH, 64), jnp.float32),
          pltpu.SemaphoreType.DMA,
          pltpu.SemaphoreType.DMA,
      ],
  )
  def k(hl_h, hr_h, src_h, dst_h, el_h, er_h, isv, idv, rbl, rbr, sem1, sem2):
    c = lax.axis_index("c")
    s = lax.axis_index("s")
    base = (s * NC + c) * PERW

    def step(t, carry):
      off = base + t * CH
      pltpu.sync_copy(src_h.at[pl.ds(off, CH)], isv)
      pltpu.sync_copy(dst_h.at[pl.ds(off, CH)], idv)
      a = pltpu.async_copy(hl_h.at[isv], rbl, sem1)
      b = pltpu.async_copy(hr_h.at[idv], rbr, sem2)
      a.wait()
      b.wait()
      pltpu.sync_copy(rbl, el_h.at[pl.ds(off, CH)])
      pltpu.sync_copy(rbr, er_h.at[pl.ds(off, CH)])
      return carry

    lax.fori_loop(0, ITERS, step, 0)

  return k(hl, hr, src, dst)


def _sc_scatter(w, dst, zinit):
  CH = 16
  PERW = E // NS
  ITERS = PERW // CH

  mesh = plsc.VectorSubcoreMesh(core_axis_name="c", subcore_axis_name="s")

  @functools.partial(
      pl.kernel,
      out_type=jax.ShapeDtypeStruct((NC * RT, W_COL), jnp.float32),
      mesh=mesh,
      compiler_params=pltpu.CompilerParams(use_tc_tiling_on_sc=False),
      scratch_types=[
          pltpu.VMEM_SHARED((RT, W_COL), jnp.float32),
          pltpu.VMEM((CH,), jnp.int32),
          pltpu.VMEM((CH, W_COL), jnp.float32),
      ],
  )
  def k(w_h, dst_h, z_h, out_h, acc, dstb, wb):
    c = lax.axis_index("c")
    s = lax.axis_index("s")
    base = c * R

    pltpu.sync_copy(z_h, acc.at[pl.ds(s * DRAIN, DRAIN)])
    plsc.subcore_barrier()

    def step(t, carry):
      off = s * PERW + t * CH
      pltpu.sync_copy(dst_h.at[pl.ds(off, CH)], dstb)
      pltpu.sync_copy(w_h.at[pl.ds(off, CH)], wb)
      l = dstb[...] - base
      inb = (l >= 0) & (l < R)
      lvec = jnp.where(inb, l, R)
      pltpu.sync_copy(wb, acc.at[lvec], add=True)
      return carry

    lax.fori_loop(0, ITERS, step, 0)
    plsc.subcore_barrier()

    pltpu.sync_copy(acc.at[pl.ds(s * DRAIN, DRAIN)],
                    out_h.at[pl.ds(c * RT + s * DRAIN, DRAIN)])

  return k(w, dst, zinit)


def _mm2_body(x_ref, wl_ref, wr_ref, o1_ref, o2_ref):
  xv = x_ref[...]
  o1_ref[...] = jnp.dot(xv, wl_ref[...], preferred_element_type=jnp.float32)
  o2_ref[...] = jnp.dot(xv, wr_ref[...], preferred_element_type=jnp.float32)


def _mm2(x, wl, wr):
  n, f = x.shape
  d = wl.shape[1]
  BN = 1000
  return pl.pallas_call(
      _mm2_body,
      grid=(n // BN,),
      in_specs=[
          pl.BlockSpec((BN, f), lambda i: (i, 0)),
          pl.BlockSpec((f, d), lambda i: (0, 0)),
          pl.BlockSpec((f, d), lambda i: (0, 0)),
      ],
      out_specs=[
          pl.BlockSpec((BN, d), lambda i: (i, 0)),
          pl.BlockSpec((BN, d), lambda i: (i, 0)),
      ],
      out_shape=[jax.ShapeDtypeStruct((n, d), jnp.float32),
                 jax.ShapeDtypeStruct((n, d), jnp.float32)],
  )(x, wl, wr)


def _lrelu(x):
  return jnp.where(x >= 0, x, 0.2 * x)


def _edge1_body(el_ref, er_ref, a_ref, w_ref):
  el = el_ref[...]
  z = _lrelu(el + er_ref[...])
  a = a_ref[...]
  parts = []
  exs = []
  for h in range(4):
    zh = z[:, 16 * h:16 * h + 16]
    eh = jnp.sum(zh * a[h, :][None, :], axis=1, keepdims=True)
    exh = jnp.exp(eh)
    exs.append(exh)
    parts.append(exh * el[:, 16 * h:16 * h + 16])
  ex = jnp.concatenate(exs, axis=1)
  pad = jnp.zeros((el.shape[0], 12), jnp.float32)
  w_ref[...] = jnp.concatenate(parts + [ex, pad], axis=1)


def _edge1(el, er, a1):
  BE = 1000
  return pl.pallas_call(
      _edge1_body,
      grid=(E // BE,),
      in_specs=[
          pl.BlockSpec((BE, 64), lambda i: (i, 0)),
          pl.BlockSpec((BE, 64), lambda i: (i, 0)),
          pl.BlockSpec((4, 16), lambda i: (0, 0)),
      ],
      out_specs=pl.BlockSpec((BE, W_COL), lambda i: (i, 0)),
      out_shape=jax.ShapeDtypeStruct((E, W_COL), jnp.float32),
  )(el, er, a1)


def _edge2_body(el_ref, er_ref, a_ref, w_ref):
  el = el_ref[...]
  z = _lrelu(el + er_ref[...])
  e = jnp.sum(z * a_ref[...], axis=1, keepdims=True)
  ex = jnp.exp(e)
  exb = jnp.broadcast_to(ex, (el.shape[0], 16))
  w_ref[...] = jnp.concatenate([ex * el, exb], axis=1)


def _edge2(el, er, a2):
  BE = 1000
  return pl.pallas_call(
      _edge2_body,
      grid=(E // BE,),
      in_specs=[
          pl.BlockSpec((BE, 64), lambda i: (i, 0)),
          pl.BlockSpec((BE, 64), lambda i: (i, 0)),
          pl.BlockSpec((1, 64), lambda i: (0, 0)),
      ],
      out_specs=pl.BlockSpec((BE, W_COL), lambda i: (i, 0)),
      out_shape=jax.ShapeDtypeStruct((E, W_COL), jnp.float32),
  )(el, er, a2)


def _layer_norm(o, g, b):
  mu = jnp.mean(o, axis=-1, keepdims=True)
  var = jnp.mean((o - mu) * (o - mu), axis=-1, keepdims=True)
  return (o - mu) / jnp.sqrt(var + 1e-5) * g + b


def _norm1_body(acc_ref, g_ref, b_ref, o_ref):
  acc = acc_ref[...]
  parts = []
  for h in range(4):
    den = acc[:, 64 + h:65 + h] + 1e-16
    parts.append(acc[:, 16 * h:16 * h + 16] / den)
  o = jnp.concatenate(parts, axis=1)
  o_ref[...] = _lrelu(_layer_norm(o, g_ref[...], b_ref[...]))


def _norm2_body(acc_ref, g_ref, b_ref, o_ref):
  acc = acc_ref[...]
  den = acc[:, 64:65] + 1e-16
  o = acc[:, :64] / den
  o_ref[...] = _lrelu(_layer_norm(o, g_ref[...], b_ref[...]))


def _norm(acc, g, b, body):
  BN = 1000
  return pl.pallas_call(
      body,
      grid=(N // BN,),
      in_specs=[
          pl.BlockSpec((BN, W_COL), lambda i: (i, 0)),
          pl.BlockSpec((1, 64), lambda i: (0, 0)),
          pl.BlockSpec((1, 64), lambda i: (0, 0)),
      ],
      out_specs=pl.BlockSpec((BN, 64), lambda i: (i, 0)),
      out_shape=jax.ShapeDtypeStruct((N, 64), jnp.float32),
  )(acc, g.reshape(1, 64), b.reshape(1, 64))


def _final_body(seg_ref, h_ref, w_ref, b_ref, o_ref):
  o_ref[0] = jnp.dot(h_ref[0], w_ref[...],
                     preferred_element_type=jnp.float32) + b_ref[...]


def _final(h2, seg_index, cls_W, cls_b):
  n_cls = cls_W.shape[1]
  n_seg = seg_index.shape[0]
  grid_spec = pltpu.PrefetchScalarGridSpec(
      num_scalar_prefetch=1,
      grid=(n_seg,),
      in_specs=[
          pl.BlockSpec((1, 1, 64), lambda i, seg: (seg[i], 0, 0)),
          pl.BlockSpec((64, n_cls), lambda i, seg: (0, 0)),
          pl.BlockSpec((1, n_cls), lambda i, seg: (0, 0)),
      ],
      out_specs=pl.BlockSpec((1, 1, n_cls), lambda i, seg: (i, 0, 0)),
  )
  out = pl.pallas_call(
      _final_body,
      grid_spec=grid_spec,
      out_shape=jax.ShapeDtypeStruct((n_seg, 1, n_cls), jnp.float32),
  )(seg_index, h2.reshape(N, 1, 64), cls_W, cls_b.reshape(1, n_cls))
  return out.reshape(n_seg, n_cls)


def _gat_layer(h, src, dst, Wl, Wr, a, zinit, edge_body):
  hl, hr = _mm2(h, Wl, Wr)
  el, er = _sc_gather(hl, hr, src, dst)
  w = edge_body(el, er, a)
  accp = _sc_scatter(w, dst, zinit)
  return jnp.concatenate([accp[:R], accp[RT:RT + R]], axis=0)


def kernel(x, adj, seg_index, Wl1, Wr1, a1, ln1_g, ln1_b,
           Wl2, Wr2, a2, ln2_g, ln2_b, cls_W, cls_b):
  xs = x[0]
  src = adj[0, 0]
  dst = adj[0, 1]
  zinit = jnp.zeros((DRAIN, W_COL), jnp.float32)

  acc1 = _gat_layer(xs, src, dst, Wl1, Wr1, a1, zinit, _edge1)
  h1 = _norm(acc1, ln1_g, ln1_b, _norm1_body)

  acc2 = _gat_layer(h1, src, dst, Wl2, Wr2, a2, zinit, _edge2)
  h2 = _norm(acc2, ln2_g, ln2_b, _norm2_body)

  return _final(h2, seg_index, cls_W, cls_b)


# speedup vs baseline: 12.6306x; 1.4811x over previous
import functools
import jax
import jax.numpy as jnp
from jax import lax
from jax.experimental import pallas as pl
from jax.experimental.pallas import tpu as pltpu
from jax.experimental.pallas import tpu_sc as plsc

N = 50000
E = 800000
NC = 2
NS = 16
R = N // NC
DRAIN = 1563
RT = NS * DRAIN
W_COL = 80


def _sc_gather(hl, hr, src, dst):
  CH = 40
  PERW = E // (NC * NS)
  ITERS = PERW // CH

  mesh = plsc.VectorSubcoreMesh(core_axis_name="c", subcore_axis_name="s")

  @functools.partial(
      pl.kernel,
      out_type=(jax.ShapeDtypeStruct((E, 64), jnp.float32),
                jax.ShapeDtypeStruct((E, 64), jnp.float32)),
      mesh=mesh,
      compiler_params=pltpu.CompilerParams(use_tc_tiling_on_sc=False),
      scratch_types=[
          pltpu.VMEM((CH,), jnp.int32),
          pltpu.VMEM((CH,), jnp.int32),
          pltpu.VMEM((CH, 64), jnp.float32),
          pltpu.VMEM((CH, 64), jnp.float32),
          pltpu.SemaphoreType.DMA,
          pltpu.SemaphoreType.DMA,
      ],
  )
  def k(hl_h, hr_h, src_h, dst_h, el_h, er_h, isv, idv, rbl, rbr, sem1, sem2):
    c = lax.axis_index("c")
    s = lax.axis_index("s")
    base = (s * NC + c) * PERW

    def step(t, carry):
      off = base + t * CH
      pltpu.sync_copy(src_h.at[pl.ds(off, CH)], isv)
      pltpu.sync_copy(dst_h.at[pl.ds(off, CH)], idv)
      a = pltpu.async_copy(hl_h.at[isv], rbl, sem1)
      b = pltpu.async_copy(hr_h.at[idv], rbr, sem2)
      a.wait()
      b.wait()
      pltpu.sync_copy(rbl, el_h.at[pl.ds(off, CH)])
      pltpu.sync_copy(rbr, er_h.at[pl.ds(off, CH)])
      return carry

    lax.fori_loop(0, ITERS, step, 0)

  return k(hl, hr, src, dst)


def _sc_scatter(w, dst, zinit):
  GRP = 4
  CH = 16
  BCH = GRP * CH
  PERW = E // NS
  ITERS = PERW // BCH
  REM = PERW - ITERS * BCH
  assert REM % CH == 0 and (ITERS * BCH) % 8 == 0

  mesh = plsc.VectorSubcoreMesh(core_axis_name="c", subcore_axis_name="s")

  @functools.partial(
      pl.kernel,
      out_type=jax.ShapeDtypeStruct((NC * RT, W_COL), jnp.float32),
      mesh=mesh,
      compiler_params=pltpu.CompilerParams(use_tc_tiling_on_sc=False),
      scratch_types=[
          pltpu.VMEM_SHARED((RT, W_COL), jnp.float32),
          pltpu.VMEM((BCH,), jnp.int32),
          pltpu.VMEM((BCH, W_COL), jnp.float32),
          pltpu.SemaphoreType.DMA,
      ],
  )
  def k(w_h, dst_h, z_h, out_h, acc, dstb, wb, sem):
    c = lax.axis_index("c")
    s = lax.axis_index("s")
    base = c * R

    pltpu.sync_copy(z_h, acc.at[pl.ds(s * DRAIN, DRAIN)])
    plsc.subcore_barrier()

    def step(t, carry):
      off = s * PERW + t * BCH
      pltpu.sync_copy(dst_h.at[pl.ds(off, BCH)], dstb)
      pltpu.sync_copy(w_h.at[pl.ds(off, BCH)], wb)
      descs = []
      for g in range(GRP):
        l = dstb[pl.ds(g * CH, CH)] - base
        inb = (l >= 0) & (l < R)
        lvec = jnp.where(inb, l, R)
        descs.append(
            pltpu.async_copy(wb.at[pl.ds(g * CH, CH)], acc.at[lvec], sem,
                             add=True))
      for d in descs:
        d.wait()
      return carry

    lax.fori_loop(0, ITERS, step, 0)

    for r in range(REM // CH):
      roff = s * PERW + ITERS * BCH + r * CH
      pltpu.sync_copy(dst_h.at[pl.ds(roff, CH)], dstb.at[pl.ds(0, CH)])
      pltpu.sync_copy(w_h.at[pl.ds(roff, CH)], wb.at[pl.ds(0, CH)])
      l = dstb[pl.ds(0, CH)] - base
      inb = (l >= 0) & (l < R)
      lvec = jnp.where(inb, l, R)
      pltpu.sync_copy(wb.at[pl.ds(0, CH)], acc.at[lvec], add=True)
    plsc.subcore_barrier()

    pltpu.sync_copy(acc.at[pl.ds(s * DRAIN, DRAIN)],
                    out_h.at[pl.ds(c * RT + s * DRAIN, DRAIN)])

  return k(w, dst, zinit)


def _mm2_body(x_ref, wl_ref, wr_ref, o1_ref, o2_ref):
  xv = x_ref[...]
  o1_ref[...] = jnp.dot(xv, wl_ref[...], preferred_element_type=jnp.float32)
  o2_ref[...] = jnp.dot(xv, wr_ref[...], preferred_element_type=jnp.float32)


def _mm2(x, wl, wr):
  n, f = x.shape
  d = wl.shape[1]
  BN = 1000
  return pl.pallas_call(
      _mm2_body,
      grid=(n // BN,),
      in_specs=[
          pl.BlockSpec((BN, f), lambda i: (i, 0)),
          pl.BlockSpec((f, d), lambda i: (0, 0)),
          pl.BlockSpec((f, d), lambda i: (0, 0)),
      ],
      out_specs=[
          pl.BlockSpec((BN, d), lambda i: (i, 0)),
          pl.BlockSpec((BN, d), lambda i: (i, 0)),
      ],
      out_shape=[jax.ShapeDtypeStruct((n, d), jnp.float32),
                 jax.ShapeDtypeStruct((n, d), jnp.float32)],
  )(x, wl, wr)


def _lrelu(x):
  return jnp.where(x >= 0, x, 0.2 * x)


def _edge1_body(el_ref, er_ref, a_ref, w_ref):
  el = el_ref[...]
  z = _lrelu(el + er_ref[...])
  a = a_ref[...]
  parts = []
  exs = []
  for h in range(4):
    zh = z[:, 16 * h:16 * h + 16]
    eh = jnp.sum(zh * a[h, :][None, :], axis=1, keepdims=True)
    exh = jnp.exp(eh)
    exs.append(exh)
    parts.append(exh * el[:, 16 * h:16 * h + 16])
  ex = jnp.concatenate(exs, axis=1)
  pad = jnp.zeros((el.shape[0], 12), jnp.float32)
  w_ref[...] = jnp.concatenate(parts + [ex, pad], axis=1)


def _edge1(el, er, a1):
  BE = 1000
  return pl.pallas_call(
      _edge1_body,
      grid=(E // BE,),
      in_specs=[
          pl.BlockSpec((BE, 64), lambda i: (i, 0)),
          pl.BlockSpec((BE, 64), lambda i: (i, 0)),
          pl.BlockSpec((4, 16), lambda i: (0, 0)),
      ],
      out_specs=pl.BlockSpec((BE, W_COL), lambda i: (i, 0)),
      out_shape=jax.ShapeDtypeStruct((E, W_COL), jnp.float32),
  )(el, er, a1)


def _edge2_body(el_ref, er_ref, a_ref, w_ref):
  el = el_ref[...]
  z = _lrelu(el + er_ref[...])
  e = jnp.sum(z * a_ref[...], axis=1, keepdims=True)
  ex = jnp.exp(e)
  exb = jnp.broadcast_to(ex, (el.shape[0], 16))
  w_ref[...] = jnp.concatenate([ex * el, exb], axis=1)


def _edge2(el, er, a2):
  BE = 1000
  return pl.pallas_call(
      _edge2_body,
      grid=(E // BE,),
      in_specs=[
          pl.BlockSpec((BE, 64), lambda i: (i, 0)),
          pl.BlockSpec((BE, 64), lambda i: (i, 0)),
          pl.BlockSpec((1, 64), lambda i: (0, 0)),
      ],
      out_specs=pl.BlockSpec((BE, W_COL), lambda i: (i, 0)),
      out_shape=jax.ShapeDtypeStruct((E, W_COL), jnp.float32),
  )(el, er, a2)


def _layer_norm(o, g, b):
  mu = jnp.mean(o, axis=-1, keepdims=True)
  var = jnp.mean((o - mu) * (o - mu), axis=-1, keepdims=True)
  return (o - mu) / jnp.sqrt(var + 1e-5) * g + b


def _norm1_body(acc_ref, g_ref, b_ref, o_ref):
  acc = acc_ref[...]
  parts = []
  for h in range(4):
    den = acc[:, 64 + h:65 + h] + 1e-16
    parts.append(acc[:, 16 * h:16 * h + 16] / den)
  o = jnp.concatenate(parts, axis=1)
  o_ref[...] = _lrelu(_layer_norm(o, g_ref[...], b_ref[...]))


def _norm2_body(acc_ref, g_ref, b_ref, o_ref):
  acc = acc_ref[...]
  den = acc[:, 64:65] + 1e-16
  o = acc[:, :64] / den
  o_ref[...] = _lrelu(_layer_norm(o, g_ref[...], b_ref[...]))


def _norm(acc, g, b, body):
  BN = 1000
  return pl.pallas_call(
      body,
      grid=(N // BN,),
      in_specs=[
          pl.BlockSpec((BN, W_COL), lambda i: (i, 0)),
          pl.BlockSpec((1, 64), lambda i: (0, 0)),
          pl.BlockSpec((1, 64), lambda i: (0, 0)),
      ],
      out_specs=pl.BlockSpec((BN, 64), lambda i: (i, 0)),
      out_shape=jax.ShapeDtypeStruct((N, 64), jnp.float32),
  )(acc, g.reshape(1, 64), b.reshape(1, 64))


def _final_body(seg_ref, h_ref, w_ref, b_ref, o_ref):
  o_ref[0] = jnp.dot(h_ref[0], w_ref[...],
                     preferred_element_type=jnp.float32) + b_ref[...]


def _final(h2, seg_index, cls_W, cls_b):
  n_cls = cls_W.shape[1]
  n_seg = seg_index.shape[0]
  grid_spec = pltpu.PrefetchScalarGridSpec(
      num_scalar_prefetch=1,
      grid=(n_seg,),
      in_specs=[
          pl.BlockSpec((1, 1, 64), lambda i, seg: (seg[i], 0, 0)),
          pl.BlockSpec((64, n_cls), lambda i, seg: (0, 0)),
          pl.BlockSpec((1, n_cls), lambda i, seg: (0, 0)),
      ],
      out_specs=pl.BlockSpec((1, 1, n_cls), lambda i, seg: (i, 0, 0)),
  )
  out = pl.pallas_call(
      _final_body,
      grid_spec=grid_spec,
      out_shape=jax.ShapeDtypeStruct((n_seg, 1, n_cls), jnp.float32),
  )(seg_index, h2.reshape(N, 1, 64), cls_W, cls_b.reshape(1, n_cls))
  return out.reshape(n_seg, n_cls)


def _gat_layer(h, src, dst, Wl, Wr, a, zinit, edge_body):
  hl, hr = _mm2(h, Wl, Wr)
  el, er = _sc_gather(hl, hr, src, dst)
  w = edge_body(el, er, a)
  accp = _sc_scatter(w, dst, zinit)
  return jnp.concatenate([accp[:R], accp[RT:RT + R]], axis=0)


def kernel(x, adj, seg_index, Wl1, Wr1, a1, ln1_g, ln1_b,
           Wl2, Wr2, a2, ln2_g, ln2_b, cls_W, cls_b):
  xs = x[0]
  src = adj[0, 0]
  dst = adj[0, 1]
  zinit = jnp.zeros((DRAIN, W_COL), jnp.float32)

  acc1 = _gat_layer(xs, src, dst, Wl1, Wr1, a1, zinit, _edge1)
  h1 = _norm(acc1, ln1_g, ln1_b, _norm1_body)

  acc2 = _gat_layer(h1, src, dst, Wl2, Wr2, a2, zinit, _edge2)
  h2 = _norm(acc2, ln2_g, ln2_b, _norm2_body)

  return _final(h2, seg_index, cls_W, cls_b)


# speedup vs baseline: 14.5264x; 1.1501x over previous
import functools
import jax
import jax.numpy as jnp
from jax import lax
from jax.experimental import pallas as pl
from jax.experimental.pallas import tpu as pltpu
from jax.experimental.pallas import tpu_sc as plsc

N = 50000
E = 800000
NC = 2
NS = 16
R = N // NC
DRAIN = 1563
RT = NS * DRAIN
W_COL = 80


def _sc_gather(hl, hr, src, dst):
  CH = 128
  NW = NC * NS
  NCHUNK = E // CH
  ITERS = NCHUNK // NW
  XTRA = NCHUNK - ITERS * NW

  mesh = plsc.VectorSubcoreMesh(core_axis_name="c", subcore_axis_name="s")

  @functools.partial(
      pl.kernel,
      out_type=(jax.ShapeDtypeStruct((E, 64), jnp.float32),
                jax.ShapeDtypeStruct((E, 64), jnp.float32)),
      mesh=mesh,
      compiler_params=pltpu.CompilerParams(use_tc_tiling_on_sc=False),
      scratch_types=[
          pltpu.VMEM((CH,), jnp.int32),
          pltpu.VMEM((CH,), jnp.int32),
          pltpu.VMEM((CH, 64), jnp.float32),
          pltpu.VMEM((CH, 64), jnp.float32),
          pltpu.SemaphoreType.DMA,
          pltpu.SemaphoreType.DMA,
      ],
  )
  def k(hl_h, hr_h, src_h, dst_h, el_h, er_h, isv, idv, rbl, rbr, sem1, sem2):
    c = lax.axis_index("c")
    s = lax.axis_index("s")
    wid = s * NC + c

    def chunk(i):
      off = i * CH
      pltpu.sync_copy(src_h.at[pl.ds(off, CH)], isv)
      pltpu.sync_copy(dst_h.at[pl.ds(off, CH)], idv)
      a = pltpu.async_copy(hl_h.at[isv], rbl, sem1)
      b = pltpu.async_copy(hr_h.at[idv], rbr, sem2)
      a.wait()
      b.wait()
      pltpu.sync_copy(rbl, el_h.at[pl.ds(off, CH)])
      pltpu.sync_copy(rbr, er_h.at[pl.ds(off, CH)])

    def step(t, carry):
      chunk(t * NW + wid)
      return carry

    lax.fori_loop(0, ITERS, step, 0)

    @pl.when(wid < XTRA)
    def _():
      chunk(ITERS * NW + wid)

  return k(hl, hr, src, dst)


def _sc_scatter(w, dst, zinit):
  GRP = 4
  CH = 16
  BCH = GRP * CH
  PERW = E // NS
  ITERS = PERW // BCH
  REM = PERW - ITERS * BCH
  assert REM % CH == 0 and (ITERS * BCH) % 8 == 0

  mesh = plsc.VectorSubcoreMesh(core_axis_name="c", subcore_axis_name="s")

  @functools.partial(
      pl.kernel,
      out_type=jax.ShapeDtypeStruct((NC * RT, W_COL), jnp.float32),
      mesh=mesh,
      compiler_params=pltpu.CompilerParams(use_tc_tiling_on_sc=False),
      scratch_types=[
          pltpu.VMEM_SHARED((RT, W_COL), jnp.float32),
          pltpu.VMEM((BCH,), jnp.int32),
          pltpu.VMEM((BCH, W_COL), jnp.float32),
          pltpu.SemaphoreType.DMA,
      ],
  )
  def k(w_h, dst_h, z_h, out_h, acc, dstb, wb, sem):
    c = lax.axis_index("c")
    s = lax.axis_index("s")
    base = c * R

    pltpu.sync_copy(z_h, acc.at[pl.ds(s * DRAIN, DRAIN)])
    plsc.subcore_barrier()

    def step(t, carry):
      off = s * PERW + t * BCH
      pltpu.sync_copy(dst_h.at[pl.ds(off, BCH)], dstb)
      pltpu.sync_copy(w_h.at[pl.ds(off, BCH)], wb)
      descs = []
      for g in range(GRP):
        l = dstb[pl.ds(g * CH, CH)] - base
        inb = (l >= 0) & (l < R)
        lvec = jnp.where(inb, l, R)
        descs.append(
            pltpu.async_copy(wb.at[pl.ds(g * CH, CH)], acc.at[lvec], sem,
                             add=True))
      for d in descs:
        d.wait()
      return carry

    lax.fori_loop(0, ITERS, step, 0)

    for r in range(REM // CH):
      roff = s * PERW + ITERS * BCH + r * CH
      pltpu.sync_copy(dst_h.at[pl.ds(roff, CH)], dstb.at[pl.ds(0, CH)])
      pltpu.sync_copy(w_h.at[pl.ds(roff, CH)], wb.at[pl.ds(0, CH)])
      l = dstb[pl.ds(0, CH)] - base
      inb = (l >= 0) & (l < R)
      lvec = jnp.where(inb, l, R)
      pltpu.sync_copy(wb.at[pl.ds(0, CH)], acc.at[lvec], add=True)
    plsc.subcore_barrier()

    pltpu.sync_copy(acc.at[pl.ds(s * DRAIN, DRAIN)],
                    out_h.at[pl.ds(c * RT + s * DRAIN, DRAIN)])

  return k(w, dst, zinit)


def _mm2_body(x_ref, wl_ref, wr_ref, o1_ref, o2_ref):
  xv = x_ref[...]
  o1_ref[...] = jnp.dot(xv, wl_ref[...], preferred_element_type=jnp.float32)
  o2_ref[...] = jnp.dot(xv, wr_ref[...], preferred_element_type=jnp.float32)


def _mm2(x, wl, wr):
  n, f = x.shape
  d = wl.shape[1]
  BN = 1000
  return pl.pallas_call(
      _mm2_body,
      grid=(n // BN,),
      in_specs=[
          pl.BlockSpec((BN, f), lambda i: (i, 0)),
          pl.BlockSpec((f, d), lambda i: (0, 0)),
          pl.BlockSpec((f, d), lambda i: (0, 0)),
      ],
      out_specs=[
          pl.BlockSpec((BN, d), lambda i: (i, 0)),
          pl.BlockSpec((BN, d), lambda i: (i, 0)),
      ],
      out_shape=[jax.ShapeDtypeStruct((n, d), jnp.float32),
                 jax.ShapeDtypeStruct((n, d), jnp.float32)],
  )(x, wl, wr)


def _lrelu(x):
  return jnp.where(x >= 0, x, 0.2 * x)


def _edge1_body(el_ref, er_ref, a_ref, w_ref):
  el = el_ref[...]
  z = _lrelu(el + er_ref[...])
  a = a_ref[...]
  parts = []
  exs = []
  for h in range(4):
    zh = z[:, 16 * h:16 * h + 16]
    eh = jnp.sum(zh * a[h, :][None, :], axis=1, keepdims=True)
    exh = jnp.exp(eh)
    exs.append(exh)
    parts.append(exh * el[:, 16 * h:16 * h + 16])
  ex = jnp.concatenate(exs, axis=1)
  pad = jnp.zeros((el.shape[0], 12), jnp.float32)
  w_ref[...] = jnp.concatenate(parts + [ex, pad], axis=1)


def _edge1(el, er, a1):
  BE = 1000
  return pl.pallas_call(
      _edge1_body,
      grid=(E // BE,),
      in_specs=[
          pl.BlockSpec((BE, 64), lambda i: (i, 0)),
          pl.BlockSpec((BE, 64), lambda i: (i, 0)),
          pl.BlockSpec((4, 16), lambda i: (0, 0)),
      ],
      out_specs=pl.BlockSpec((BE, W_COL), lambda i: (i, 0)),
      out_shape=jax.ShapeDtypeStruct((E, W_COL), jnp.float32),
  )(el, er, a1)


def _edge2_body(el_ref, er_ref, a_ref, w_ref):
  el = el_ref[...]
  z = _lrelu(el + er_ref[...])
  e = jnp.sum(z * a_ref[...], axis=1, keepdims=True)
  ex = jnp.exp(e)
  exb = jnp.broadcast_to(ex, (el.shape[0], 16))
  w_ref[...] = jnp.concatenate([ex * el, exb], axis=1)


def _edge2(el, er, a2):
  BE = 1000
  return pl.pallas_call(
      _edge2_body,
      grid=(E // BE,),
      in_specs=[
          pl.BlockSpec((BE, 64), lambda i: (i, 0)),
          pl.BlockSpec((BE, 64), lambda i: (i, 0)),
          pl.BlockSpec((1, 64), lambda i: (0, 0)),
      ],
      out_specs=pl.BlockSpec((BE, W_COL), lambda i: (i, 0)),
      out_shape=jax.ShapeDtypeStruct((E, W_COL), jnp.float32),
  )(el, er, a2)


def _layer_norm(o, g, b):
  mu = jnp.mean(o, axis=-1, keepdims=True)
  var = jnp.mean((o - mu) * (o - mu), axis=-1, keepdims=True)
  return (o - mu) / jnp.sqrt(var + 1e-5) * g + b


def _norm1_body(acc_ref, g_ref, b_ref, o_ref):
  acc = acc_ref[...]
  parts = []
  for h in range(4):
    den = acc[:, 64 + h:65 + h] + 1e-16
    parts.append(acc[:, 16 * h:16 * h + 16] / den)
  o = jnp.concatenate(parts, axis=1)
  o_ref[...] = _lrelu(_layer_norm(o, g_ref[...], b_ref[...]))


def _norm2_body(acc_ref, g_ref, b_ref, o_ref):
  acc = acc_ref[...]
  den = acc[:, 64:65] + 1e-16
  o = acc[:, :64] / den
  o_ref[...] = _lrelu(_layer_norm(o, g_ref[...], b_ref[...]))


def _norm(acc, g, b, body):
  BN = 1000
  return pl.pallas_call(
      body,
      grid=(N // BN,),
      in_specs=[
          pl.BlockSpec((BN, W_COL), lambda i: (i, 0)),
          pl.BlockSpec((1, 64), lambda i: (0, 0)),
          pl.BlockSpec((1, 64), lambda i: (0, 0)),
      ],
      out_specs=pl.BlockSpec((BN, 64), lambda i: (i, 0)),
      out_shape=jax.ShapeDtypeStruct((N, 64), jnp.float32),
  )(acc, g.reshape(1, 64), b.reshape(1, 64))


def _final_body(seg_ref, h_ref, w_ref, b_ref, o_ref):
  o_ref[0] = jnp.dot(h_ref[0], w_ref[...],
                     preferred_element_type=jnp.float32) + b_ref[...]


def _final(h2, seg_index, cls_W, cls_b):
  n_cls = cls_W.shape[1]
  n_seg = seg_index.shape[0]
  grid_spec = pltpu.PrefetchScalarGridSpec(
      num_scalar_prefetch=1,
      grid=(n_seg,),
      in_specs=[
          pl.BlockSpec((1, 1, 64), lambda i, seg: (seg[i], 0, 0)),
          pl.BlockSpec((64, n_cls), lambda i, seg: (0, 0)),
          pl.BlockSpec((1, n_cls), lambda i, seg: (0, 0)),
      ],
      out_specs=pl.BlockSpec((1, 1, n_cls), lambda i, seg: (i, 0, 0)),
  )
  out = pl.pallas_call(
      _final_body,
      grid_spec=grid_spec,
      out_shape=jax.ShapeDtypeStruct((n_seg, 1, n_cls), jnp.float32),
  )(seg_index, h2.reshape(N, 1, 64), cls_W, cls_b.reshape(1, n_cls))
  return out.reshape(n_seg, n_cls)


def _gat_layer(h, src, dst, Wl, Wr, a, zinit, edge_body):
  hl, hr = _mm2(h, Wl, Wr)
  el, er = _sc_gather(hl, hr, src, dst)
  w = edge_body(el, er, a)
  accp = _sc_scatter(w, dst, zinit)
  return jnp.concatenate([accp[:R], accp[RT:RT + R]], axis=0)


def kernel(x, adj, seg_index, Wl1, Wr1, a1, ln1_g, ln1_b,
           Wl2, Wr2, a2, ln2_g, ln2_b, cls_W, cls_b):
  xs = x[0]
  src = adj[0, 0]
  dst = adj[0, 1]
  zinit = jnp.zeros((DRAIN, W_COL), jnp.float32)

  acc1 = _gat_layer(xs, src, dst, Wl1, Wr1, a1, zinit, _edge1)
  h1 = _norm(acc1, ln1_g, ln1_b, _norm1_body)

  acc2 = _gat_layer(h1, src, dst, Wl2, Wr2, a2, zinit, _edge2)
  h2 = _norm(acc2, ln2_g, ln2_b, _norm2_body)

  return _final(h2, seg_index, cls_W, cls_b)
